# baseline (speedup 1.0000x reference)
import functools

import jax
import jax.numpy as jnp
from jax import lax
from jax.experimental import pallas as pl
from jax.experimental.pallas import tpu as pltpu
from jax.experimental.pallas import tpu_sc as plsc

N_NODES = 10000
N_EDGES = 320000
D = 128
G = 64

NC = 2
NS = 16
NW = NC * NS

N_PAD = 10240
BROWS = N_PAD // NW
TRASH = BROWS
GC = 128
EPAD = N_EDGES + GC
EPW = N_EDGES // NW
CP = 80

_mesh = plsc.VectorSubcoreMesh(core_axis_name="c", subcore_axis_name="s")

R = 1000
NBLK = N_NODES // R
RB = 2000
NBE = N_EDGES // RB


def _bucket_of(col):
    return lax.shift_right_logical(col * 6554, 21)


def _tc_counts_body(col_ref, brow_ref, blo_ref, bhi_ref, cnt_ref):
    i = pl.program_id(0)
    bkt = _bucket_of(col_ref[...])
    gi = lax.broadcasted_iota(jnp.int32, (RB, NW), 1)
    p = (bkt == gi).astype(jnp.float32)
    cs = jnp.sum(p, axis=0, keepdims=True)

    @pl.when(i == 0)
    def _():
        cnt_ref[...] = cs

    @pl.when(i > 0)
    def _():
        cnt_ref[...] += cs

    @pl.when(i == NBE - 1)
    def _():
        cnt = cnt_ref[...]
        ir = lax.broadcasted_iota(jnp.int32, (NW, NW), 0)
        ic = lax.broadcasted_iota(jnp.int32, (NW, NW), 1)
        tri = (ir < ic).astype(jnp.float32)
        eye = (ir == ic).astype(jnp.float32)
        dn = (((1,), (1,)), ((), ()))
        bases = lax.dot_general(cnt, tri, dn,
                                preferred_element_type=jnp.float32, precision=lax.Precision.HIGHEST)
        brow_ref[...] = bases
        bcol = lax.dot_general(eye, bases, dn,
                               preferred_element_type=jnp.float32, precision=lax.Precision.HIGHEST)
        hcol = lax.dot_general(eye, bases + cnt, dn,
                               preferred_element_type=jnp.float32, precision=lax.Precision.HIGHEST)
        blo_ref[...] = jnp.broadcast_to(bcol, (NW, 16)).astype(jnp.int32)
        bhi_ref[...] = jnp.broadcast_to(hcol, (NW, 16)).astype(jnp.int32)


_tc_counts = pl.pallas_call(
    _tc_counts_body,
    grid=(NBE,),
    in_specs=[pl.BlockSpec((RB, 1), lambda i: (i, 0))],
    out_specs=[pl.BlockSpec((1, NW), lambda i: (0, 0)),
               pl.BlockSpec((NW, 16), lambda i: (0, 0)),
               pl.BlockSpec((NW, 16), lambda i: (0, 0))],
    out_shape=[jax.ShapeDtypeStruct((1, NW), jnp.float32),
               jax.ShapeDtypeStruct((NW, 16), jnp.int32),
               jax.ShapeDtypeStruct((NW, 16), jnp.int32)],
    scratch_shapes=[pltpu.VMEM((1, NW), jnp.float32)],
)


def _tc_pos_body(col_ref, brow_ref, pos_ref, carry_ref):
    i = pl.program_id(0)

    @pl.when(i == 0)
    def _():
        carry_ref[...] = jnp.zeros((1, NW), jnp.float32)

    bkt = _bucket_of(col_ref[...])
    gi = lax.broadcasted_iota(jnp.int32, (RB, NW), 1)
    p = (bkt == gi).astype(jnp.float32)
    ri = lax.broadcasted_iota(jnp.int32, (RB, NW), 0)
    pr = p
    sh = 1
    while sh < RB:
        pr = pr + jnp.where(ri >= sh, pltpu.roll(pr, sh, 0), 0.0)
        sh *= 2
    pr = pr - p
    offs = carry_ref[...] + brow_ref[...]
    posf = jnp.sum(p * (pr + offs), axis=1, keepdims=True)
    pos_ref[...] = posf.astype(jnp.int32)
    carry_ref[...] += jnp.sum(p, axis=0, keepdims=True)


_tc_pos = pl.pallas_call(
    _tc_pos_body,
    grid=(NBE,),
    in_specs=[pl.BlockSpec((RB, 1), lambda i: (i, 0)),
              pl.BlockSpec((1, NW), lambda i: (0, 0))],
    out_specs=pl.BlockSpec((RB, 1), lambda i: (i, 0)),
    out_shape=jax.ShapeDtypeStruct((N_EDGES, 1), jnp.int32),
    scratch_shapes=[pltpu.VMEM((1, NW), jnp.float32)],
)


@functools.partial(
    pl.kernel,
    out_type=(jax.ShapeDtypeStruct((EPAD,), jnp.int32),
              jax.ShapeDtypeStruct((EPAD,), jnp.int32)),
    mesh=_mesh,
    scratch_types=[
        pltpu.VMEM((CP,), jnp.int32),
        pltpu.VMEM((CP,), jnp.int32),
        pltpu.VMEM((CP,), jnp.int32),
        pltpu.VMEM((GC,), jnp.int32),
    ],
)
def _place_sc(row_hbm, col_hbm, pos_hbm, brow_hbm, bcol_hbm,
              rbuf, cbuf, pbuf, padbuf):
    c = lax.axis_index("c")
    s = lax.axis_index("s")
    w = s * NC + c
    base = w * EPW

    def body(j, _):
        off = base + j * CP
        pltpu.sync_copy(row_hbm.at[pl.ds(off, CP)], rbuf)
        pltpu.sync_copy(col_hbm.at[pl.ds(off, CP)], cbuf)
        pltpu.sync_copy(pos_hbm.at[pl.ds(off, CP)], pbuf)
        pltpu.sync_copy(rbuf, brow_hbm.at[pbuf])
        pltpu.sync_copy(cbuf, bcol_hbm.at[pbuf])
        return 0

    lax.fori_loop(0, EPW // CP, body, 0)

    @pl.when(w == 0)
    def _():
        zv = jnp.zeros((16,), jnp.int32)
        tv = jnp.full((16,), N_PAD, jnp.int32)
        for k in range(GC // 16):
            padbuf[pl.ds(k * 16, 16)] = zv
        pltpu.sync_copy(padbuf, brow_hbm.at[pl.ds(N_EDGES, GC)])
        for k in range(GC // 16):
            padbuf[pl.ds(k * 16, 16)] = tv
        pltpu.sync_copy(padbuf, bcol_hbm.at[pl.ds(N_EDGES, GC)])


def _drain_bounds(blo_hbm, bhi_hbm, w, bbuf):
    pltpu.sync_copy(blo_hbm.at[w], bbuf)
    b0 = bbuf[0, pl.ds(0, 16)][0]
    pltpu.sync_copy(bhi_hbm.at[w], bbuf)
    b1 = bbuf[0, pl.ds(0, 16)][0]
    start = lax.shift_left(lax.shift_right_logical(b0, 7), 7)
    nchunk = lax.shift_right_logical(b1 - start + (GC - 1), 7)
    return start, nchunk


DROWS = BROWS // 16 + 1


@functools.partial(
    pl.kernel,
    out_type=jax.ShapeDtypeStruct((NW, DROWS, 16), jnp.float32),
    mesh=_mesh,
    scratch_types=[
        pltpu.VMEM((GC,), jnp.int32),
        pltpu.VMEM((1, 16), jnp.int32),
        pltpu.VMEM((DROWS + 7, 16), jnp.float32),
    ],
)
def _deg_sc(bcol_hbm, blo_hbm, bhi_hbm, out_hbm, cbuf, bbuf, degl):
    c = lax.axis_index("c")
    s = lax.axis_index("s")
    w = s * NC + c
    lo = w * BROWS
    iota_f = lax.iota(jnp.int32, 16).astype(jnp.float32)
    lo16 = jnp.full((16,), lo, jnp.int32)
    neg16 = jnp.full((16,), -1, jnp.int32)
    brows16 = jnp.full((16,), BROWS, jnp.int32)

    def _zrow(i, _):
        degl[i, pl.ds(0, 16)] = jnp.zeros((16,), jnp.float32)
        return 0

    lax.fori_loop(0, DROWS + 7, _zrow, 0)

    start, nchunk = _drain_bounds(blo_hbm, bhi_hbm, w, bbuf)

    def drain(i, _):
        off = pl.multiple_of(start + i * GC, GC)
        pltpu.sync_copy(bcol_hbm.at[pl.ds(off, GC)], cbuf)

        def vb(v, _):
            cv = cbuf[pl.ds(v * 16, 16)]
            ck = jnp.minimum(jnp.maximum(cv - lo16, neg16), brows16) + 1
            for l in range(16):
                cl = ck[l]
                r = lax.shift_right_logical(cl, 4)
                lane_f = jnp.full((16,), (cl & 15).astype(jnp.float32))
                oh = jnp.maximum(1.0 - jnp.abs(iota_f - lane_f), 0.0)
                degl[r, pl.ds(0, 16)] += oh
            return 0

        lax.fori_loop(0, GC // 16, vb, 0)
        return 0

    lax.fori_loop(0, nchunk, drain, 0)
    pltpu.sync_copy(degl.at[pl.ds(0, DROWS), :], out_hbm.at[w])


@functools.partial(
    pl.kernel,
    out_type=jax.ShapeDtypeStruct((N_PAD, D), jnp.float32),
    mesh=_mesh,
    scratch_types=[
        pltpu.VMEM((GC,), jnp.int32),
        pltpu.VMEM((GC,), jnp.int32),
        pltpu.VMEM((1, 16), jnp.int32),
        pltpu.VMEM((GC, D), jnp.float32),
        pltpu.VMEM((BROWS + 8, D), jnp.float32),
        pltpu.SemaphoreType.DMA,
    ],
)
def _spmm_sc(bcol_hbm, brow_hbm, blo_hbm, bhi_hbm, t_hbm, out_hbm,
             cbuf, rbuf, bbuf, rows_b, acc, sem):
    c = lax.axis_index("c")
    s = lax.axis_index("s")
    w = s * NC + c
    lo = w * BROWS
    lo16 = jnp.full((16,), lo, jnp.int32)
    neg16 = jnp.full((16,), -1, jnp.int32)
    brows16 = jnp.full((16,), BROWS, jnp.int32)

    def _zrow(i, _):
        for f in range(D // 16):
            acc[i, pl.ds(f * 16, 16)] = jnp.zeros((16,), jnp.float32)
        return 0

    lax.fori_loop(0, BROWS + 8, _zrow, 0)

    start, nchunk = _drain_bounds(blo_hbm, bhi_hbm, w, bbuf)

    def drain(i, _):
        off = pl.multiple_of(start + i * GC, GC)
        pltpu.sync_copy(bcol_hbm.at[pl.ds(off, GC)], cbuf)
        pltpu.sync_copy(brow_hbm.at[pl.ds(off, GC)], rbuf)
        pltpu.async_copy(t_hbm.at[rbuf], rows_b, sem).wait()

        def vb(v, _):
            cv = cbuf[pl.ds(v * 16, 16)]
            ck = jnp.minimum(jnp.maximum(cv - lo16, neg16), brows16) + 1
            for l in range(16):
                cl = ck[l]
                for f in range(D // 16):
                    acc[cl, pl.ds(f * 16, 16)] += rows_b[v * 16 + l,
                                                         pl.ds(f * 16, 16)]
            return 0

        lax.fori_loop(0, GC // 16, vb, 0)
        return 0

    lax.fori_loop(0, nchunk, drain, 0)
    pltpu.sync_copy(acc.at[pl.ds(1, BROWS), :],
                    out_hbm.at[pl.ds(lo, BROWS), :])


def _dinv_of(deg_ref):
    d = deg_ref[...] + 1.0
    y = lax.rsqrt(d)
    y = y * (1.5 - 0.5 * d * y * y)
    y = y * (1.5 - 0.5 * d * y * y)
    return y


def _mm(a, w):
    return jnp.dot(a, w, preferred_element_type=jnp.float32)


def _tc_first_body(x_ref, w_ref, deg_ref, out_ref):
    dinv = _dinv_of(deg_ref)
    t = _mm(x_ref[...], w_ref[...])
    out_ref[...] = t * dinv


def _tc_mid_body(a_ref, tp_ref, deg_ref, b_ref, w_ref, out_ref):
    dinv = _dinv_of(deg_ref)
    a = a_ref[...] + tp_ref[...]
    h = jnp.maximum(a * dinv + b_ref[...], 0.0)
    out_ref[...] = _mm(h, w_ref[...]) * dinv


def _tc_pool_body(a_ref, tp_ref, deg_ref, b_ref, wl_ref, bl_ref, batch_ref,
                  out_ref, cnt_ref):
    i = pl.program_id(0)
    dinv = _dinv_of(deg_ref)
    h = (a_ref[...] + tp_ref[...]) * dinv + b_ref[...]
    y = _mm(h, wl_ref[...])
    gids = lax.broadcasted_iota(jnp.int32, (R, G), 1)
    p = (batch_ref[...] == gids).astype(jnp.float32)
    dn = (((0,), (0,)), ((), ()))
    ysum = lax.dot_general(p, y, dn, preferred_element_type=jnp.float32, precision=lax.Precision.HIGHEST)
    csum = lax.dot_general(p, jnp.ones((R, 1), jnp.float32), dn,
                           preferred_element_type=jnp.float32, precision=lax.Precision.HIGHEST)

    @pl.when(i == 0)
    def _():
        out_ref[...] = ysum
        cnt_ref[...] = csum

    @pl.when(i > 0)
    def _():
        out_ref[...] += ysum
        cnt_ref[...] += csum

    @pl.when(i == NBLK - 1)
    def _():
        out_ref[...] = out_ref[...] / jnp.maximum(cnt_ref[...], 1.0) + bl_ref[...]


_deg_spec = pl.BlockSpec((R, 1), lambda i: (i, 0))
_w_spec = pl.BlockSpec((D, D), lambda i: (0, 0))
_b_spec = pl.BlockSpec((1, D), lambda i: (0, 0))
_row_spec = pl.BlockSpec((R, D), lambda i: (i, 0))

_tc_first = pl.pallas_call(
    _tc_first_body,
    grid=(NBLK,),
    in_specs=[_row_spec, _w_spec, _deg_spec],
    out_specs=_row_spec,
    out_shape=jax.ShapeDtypeStruct((N_NODES, D), jnp.float32),
)

_tc_mid = pl.pallas_call(
    _tc_mid_body,
    grid=(NBLK,),
    in_specs=[_row_spec, _row_spec, _deg_spec, _b_spec, _w_spec],
    out_specs=_row_spec,
    out_shape=jax.ShapeDtypeStruct((N_NODES, D), jnp.float32),
)

_tc_pool = pl.pallas_call(
    _tc_pool_body,
    grid=(NBLK,),
    in_specs=[_row_spec, _row_spec, _deg_spec, _b_spec,
              pl.BlockSpec((D, 1), lambda i: (0, 0)),
              pl.BlockSpec((1, 1), lambda i: (0, 0)),
              pl.BlockSpec((R, 1), lambda i: (i, 0))],
    out_specs=pl.BlockSpec((G, 1), lambda i: (0, 0)),
    out_shape=jax.ShapeDtypeStruct((G, 1), jnp.float32),
    scratch_shapes=[pltpu.VMEM((G, 1), jnp.float32)],
)


def kernel(x, edge_index, batch, W1, b1, W2, b2, W3, b3, Wl, bl):
    row = edge_index[0]
    col = edge_index[1]
    col2 = col.reshape(N_EDGES, 1)
    bases_row, blo, bhi = _tc_counts(col2)
    pos = _tc_pos(col2, bases_row).reshape(N_EDGES)
    brow, bcol = _place_sc(row, col, pos)
    blo = blo.reshape(NW, 1, 16)
    bhi = bhi.reshape(NW, 1, 16)
    deg_r = _deg_sc(bcol, blo, bhi)
    deg3 = deg_r.reshape(NW, DROWS * 16)[:, 1:BROWS + 1]
    deg3 = deg3.reshape(N_PAD, 1)[:N_NODES]
    t1 = _tc_first(x, W1, deg3)
    a1 = _spmm_sc(bcol, brow, blo, bhi, t1)
    t2 = _tc_mid(a1, t1, deg3, b1.reshape(1, D), W2)
    a2 = _spmm_sc(bcol, brow, blo, bhi, t2)
    t3 = _tc_mid(a2, t2, deg3, b2.reshape(1, D), W3)
    a3 = _spmm_sc(bcol, brow, blo, bhi, t3)
    out = _tc_pool(a3, t3, deg3, b3.reshape(1, D), Wl,
                   bl.reshape(1, 1), batch.reshape(N_NODES, 1))
    return out


# speedup vs baseline: 1.0561x; 1.0561x over previous
import functools

import jax
import jax.numpy as jnp
from jax import lax
from jax.experimental import pallas as pl
from jax.experimental.pallas import tpu as pltpu
from jax.experimental.pallas import tpu_sc as plsc

N_NODES = 10000
N_EDGES = 320000
D = 128
G = 64

NC = 2
NS = 16
NW = NC * NS

N_PAD = 10240
BROWS = N_PAD // NW
TRASH = BROWS
GC = 128
SG = 512
EPADDED = 327680
PADE = EPADDED - N_EDGES
EPW = EPADDED // NW

_mesh = plsc.VectorSubcoreMesh(core_axis_name="c", subcore_axis_name="s")

R = 1000
NBLK = N_NODES // R
RB = 2000
NBE = N_EDGES // RB


def _bucket_of(col):
    return lax.shift_right_logical(col * 6554, 21)


def _tc_counts_body(col_ref, brow_ref, blo_ref, bhi_ref, cnt_ref):
    i = pl.program_id(0)
    bkt = _bucket_of(col_ref[...])
    gi = lax.broadcasted_iota(jnp.int32, (RB, NW), 1)
    p = (bkt == gi).astype(jnp.float32)
    cs = jnp.sum(p, axis=0, keepdims=True)

    @pl.when(i == 0)
    def _():
        cnt_ref[...] = cs

    @pl.when(i > 0)
    def _():
        cnt_ref[...] += cs

    @pl.when(i == NBE - 1)
    def _():
        cnt = cnt_ref[...]
        ir = lax.broadcasted_iota(jnp.int32, (NW, NW), 0)
        ic = lax.broadcasted_iota(jnp.int32, (NW, NW), 1)
        tri = (ir < ic).astype(jnp.float32)
        eye = (ir == ic).astype(jnp.float32)
        dn = (((1,), (1,)), ((), ()))
        bases = lax.dot_general(cnt, tri, dn,
                                preferred_element_type=jnp.float32, precision=lax.Precision.HIGHEST)
        brow_ref[...] = bases
        bcol = lax.dot_general(eye, bases, dn,
                               preferred_element_type=jnp.float32, precision=lax.Precision.HIGHEST)
        hcol = lax.dot_general(eye, bases + cnt, dn,
                               preferred_element_type=jnp.float32, precision=lax.Precision.HIGHEST)
        blo_ref[...] = jnp.broadcast_to(bcol, (NW, 16)).astype(jnp.int32)
        bhi_ref[...] = jnp.broadcast_to(hcol, (NW, 16)).astype(jnp.int32)


_tc_counts = pl.pallas_call(
    _tc_counts_body,
    grid=(NBE,),
    in_specs=[pl.BlockSpec((RB, 1), lambda i: (i, 0))],
    out_specs=[pl.BlockSpec((1, NW), lambda i: (0, 0)),
               pl.BlockSpec((NW, 16), lambda i: (0, 0)),
               pl.BlockSpec((NW, 16), lambda i: (0, 0))],
    out_shape=[jax.ShapeDtypeStruct((1, NW), jnp.float32),
               jax.ShapeDtypeStruct((NW, 16), jnp.int32),
               jax.ShapeDtypeStruct((NW, 16), jnp.int32)],
    scratch_shapes=[pltpu.VMEM((1, NW), jnp.float32)],
)


def _tc_pos_body(col_ref, brow_ref, pos_ref, carry_ref):
    i = pl.program_id(0)

    @pl.when(i == 0)
    def _():
        carry_ref[...] = jnp.zeros((1, NW), jnp.float32)

    bkt = _bucket_of(col_ref[...])
    gi = lax.broadcasted_iota(jnp.int32, (RB, NW), 1)
    p = (bkt == gi).astype(jnp.float32)
    ri = lax.broadcasted_iota(jnp.int32, (RB, NW), 0)
    pr = p
    sh = 1
    while sh < RB:
        pr = pr + jnp.where(ri >= sh, pltpu.roll(pr, sh, 0), 0.0)
        sh *= 2
    pr = pr - p
    offs = carry_ref[...] + brow_ref[...]
    posf = jnp.sum(p * (pr + offs), axis=1, keepdims=True)
    pos_ref[...] = posf.astype(jnp.int32)
    carry_ref[...] += jnp.sum(p, axis=0, keepdims=True)


_tc_pos = pl.pallas_call(
    _tc_pos_body,
    grid=(NBE,),
    in_specs=[pl.BlockSpec((RB, 1), lambda i: (i, 0)),
              pl.BlockSpec((1, NW), lambda i: (0, 0))],
    out_specs=pl.BlockSpec((RB, 1), lambda i: (i, 0)),
    out_shape=jax.ShapeDtypeStruct((N_EDGES, 1), jnp.int32),
    scratch_shapes=[pltpu.VMEM((1, NW), jnp.float32)],
)


SPR = 8


@functools.partial(
    pl.kernel,
    out_type=(jax.ShapeDtypeStruct((EPADDED,), jnp.int32),
              jax.ShapeDtypeStruct((EPADDED,), jnp.int32)),
    mesh=_mesh,
    scratch_types=[
        pltpu.VMEM((SPR, GC), jnp.int32),
        pltpu.VMEM((SPR, GC), jnp.int32),
        pltpu.VMEM((SPR, GC), jnp.int32),
        pltpu.SemaphoreType.DMA,
    ],
)
def _place_sc(row_hbm, col_hbm, pos_hbm, brow_hbm, bcol_hbm,
              rbuf, cbuf, pbuf, sem):
    c = lax.axis_index("c")
    s = lax.axis_index("s")
    w = s * NC + c
    base = w * (EPW // GC)

    def body(j, _):
        ro = base + j * SPR
        pltpu.sync_copy(row_hbm.at[pl.ds(ro, SPR), :], rbuf)
        pltpu.sync_copy(col_hbm.at[pl.ds(ro, SPR), :], cbuf)
        pltpu.sync_copy(pos_hbm.at[pl.ds(ro, SPR), :], pbuf)
        handles = []
        for k in range(SPR):
            ps = pbuf.at[k]
            handles.append(pltpu.async_copy(rbuf.at[k], brow_hbm.at[ps], sem))
            handles.append(pltpu.async_copy(cbuf.at[k], bcol_hbm.at[ps], sem))
        for h in handles:
            h.wait()
        return 0

    lax.fori_loop(0, EPW // (SPR * GC), body, 0)


def _drain_bounds(blo_hbm, bhi_hbm, w, bbuf, shift=7):
    pltpu.sync_copy(blo_hbm.at[w], bbuf)
    b0 = bbuf[0, pl.ds(0, 16)][0]
    pltpu.sync_copy(bhi_hbm.at[w], bbuf)
    b1 = bbuf[0, pl.ds(0, 16)][0]
    start = lax.shift_left(lax.shift_right_logical(b0, 7), 7)
    nchunk = lax.shift_right_logical(b1 - start + (1 << shift) - 1, shift)
    return start, nchunk


DROWS = BROWS // 16 + 1


@functools.partial(
    pl.kernel,
    out_type=jax.ShapeDtypeStruct((NW, DROWS, 16), jnp.float32),
    mesh=_mesh,
    scratch_types=[
        pltpu.VMEM((GC,), jnp.int32),
        pltpu.VMEM((1, 16), jnp.int32),
        pltpu.VMEM((DROWS + 7, 16), jnp.float32),
    ],
)
def _deg_sc(bcol_hbm, blo_hbm, bhi_hbm, out_hbm, cbuf, bbuf, degl):
    c = lax.axis_index("c")
    s = lax.axis_index("s")
    w = s * NC + c
    lo = w * BROWS
    iota_f = lax.iota(jnp.int32, 16).astype(jnp.float32)
    lo16 = jnp.full((16,), lo, jnp.int32)
    neg16 = jnp.full((16,), -1, jnp.int32)
    brows16 = jnp.full((16,), BROWS, jnp.int32)

    def _zrow(i, _):
        degl[i, pl.ds(0, 16)] = jnp.zeros((16,), jnp.float32)
        return 0

    lax.fori_loop(0, DROWS + 7, _zrow, 0)

    start, nchunk = _drain_bounds(blo_hbm, bhi_hbm, w, bbuf)

    def drain(i, _):
        off = pl.multiple_of(start + i * GC, GC)
        pltpu.sync_copy(bcol_hbm.at[pl.ds(off, GC)], cbuf)

        def vb(v, _):
            cv = cbuf[pl.ds(v * 16, 16)]
            ck = jnp.minimum(jnp.maximum(cv - lo16, neg16), brows16) + 1
            for l in range(16):
                cl = ck[l]
                r = lax.shift_right_logical(cl, 4)
                lane_f = jnp.full((16,), (cl & 15).astype(jnp.float32))
                oh = jnp.maximum(1.0 - jnp.abs(iota_f - lane_f), 0.0)
                degl[r, pl.ds(0, 16)] += oh
            return 0

        lax.fori_loop(0, GC // 16, vb, 0)
        return 0

    lax.fori_loop(0, nchunk, drain, 0)
    pltpu.sync_copy(degl.at[pl.ds(0, DROWS), :], out_hbm.at[w])


@functools.partial(
    pl.kernel,
    out_type=jax.ShapeDtypeStruct((N_PAD, D), jnp.float32),
    mesh=_mesh,
    scratch_types=[
        pltpu.VMEM((SG,), jnp.int32),
        pltpu.VMEM((SG,), jnp.int32),
        pltpu.VMEM((1, 16), jnp.int32),
        pltpu.VMEM((SG, D), jnp.float32),
        pltpu.VMEM((BROWS + 8, D), jnp.float32),
        pltpu.SemaphoreType.DMA,
    ],
)
def _spmm_sc(bcol_hbm, brow_hbm, blo_hbm, bhi_hbm, t_hbm, out_hbm,
             cbuf, rbuf, bbuf, rows_b, acc, sem):
    c = lax.axis_index("c")
    s = lax.axis_index("s")
    w = s * NC + c
    lo = w * BROWS
    lo16 = jnp.full((16,), lo, jnp.int32)
    neg16 = jnp.full((16,), -1, jnp.int32)
    brows16 = jnp.full((16,), BROWS, jnp.int32)

    def _zrow(i, _):
        for f in range(D // 16):
            acc[i, pl.ds(f * 16, 16)] = jnp.zeros((16,), jnp.float32)
        return 0

    lax.fori_loop(0, BROWS + 8, _zrow, 0)

    start, nchunk = _drain_bounds(blo_hbm, bhi_hbm, w, bbuf, shift=9)

    def drain(i, _):
        off = pl.multiple_of(start + i * SG, GC)
        pltpu.sync_copy(bcol_hbm.at[pl.ds(off, SG)], cbuf)
        pltpu.sync_copy(brow_hbm.at[pl.ds(off, SG)], rbuf)
        handles = [
            pltpu.async_copy(t_hbm.at[rbuf.at[pl.ds(k * GC, GC)]],
                             rows_b.at[pl.ds(k * GC, GC), :], sem)
            for k in range(SG // GC)
        ]
        for h in handles:
            h.wait()

        def vb(v, _):
            cv = cbuf[pl.ds(v * 16, 16)]
            ck = jnp.minimum(jnp.maximum(cv - lo16, neg16), brows16) + 1
            for l in range(16):
                cl = ck[l]
                for f in range(D // 16):
                    acc[cl, pl.ds(f * 16, 16)] += rows_b[v * 16 + l,
                                                         pl.ds(f * 16, 16)]
            return 0

        lax.fori_loop(0, SG // 16, vb, 0)
        return 0

    lax.fori_loop(0, nchunk, drain, 0)
    pltpu.sync_copy(acc.at[pl.ds(1, BROWS), :],
                    out_hbm.at[pl.ds(lo, BROWS), :])


def _dinv_of(deg_ref):
    d = deg_ref[...] + 1.0
    y = lax.rsqrt(d)
    y = y * (1.5 - 0.5 * d * y * y)
    y = y * (1.5 - 0.5 * d * y * y)
    return y


def _mm(a, w):
    return jnp.dot(a, w, preferred_element_type=jnp.float32)


def _tc_first_body(x_ref, w_ref, deg_ref, out_ref):
    dinv = _dinv_of(deg_ref)
    t = _mm(x_ref[...], w_ref[...])
    out_ref[...] = t * dinv


def _tc_mid_body(a_ref, tp_ref, deg_ref, b_ref, w_ref, out_ref):
    dinv = _dinv_of(deg_ref)
    a = a_ref[...] + tp_ref[...]
    h = jnp.maximum(a * dinv + b_ref[...], 0.0)
    out_ref[...] = _mm(h, w_ref[...]) * dinv


def _tc_pool_body(a_ref, tp_ref, deg_ref, b_ref, wl_ref, bl_ref, batch_ref,
                  out_ref, cnt_ref):
    i = pl.program_id(0)
    dinv = _dinv_of(deg_ref)
    h = (a_ref[...] + tp_ref[...]) * dinv + b_ref[...]
    y = _mm(h, wl_ref[...])
    gids = lax.broadcasted_iota(jnp.int32, (R, G), 1)
    p = (batch_ref[...] == gids).astype(jnp.float32)
    dn = (((0,), (0,)), ((), ()))
    ysum = lax.dot_general(p, y, dn, preferred_element_type=jnp.float32, precision=lax.Precision.HIGHEST)
    csum = lax.dot_general(p, jnp.ones((R, 1), jnp.float32), dn,
                           preferred_element_type=jnp.float32, precision=lax.Precision.HIGHEST)

    @pl.when(i == 0)
    def _():
        out_ref[...] = ysum
        cnt_ref[...] = csum

    @pl.when(i > 0)
    def _():
        out_ref[...] += ysum
        cnt_ref[...] += csum

    @pl.when(i == NBLK - 1)
    def _():
        out_ref[...] = out_ref[...] / jnp.maximum(cnt_ref[...], 1.0) + bl_ref[...]


_deg_spec = pl.BlockSpec((R, 1), lambda i: (i, 0))
_w_spec = pl.BlockSpec((D, D), lambda i: (0, 0))
_b_spec = pl.BlockSpec((1, D), lambda i: (0, 0))
_row_spec = pl.BlockSpec((R, D), lambda i: (i, 0))

_tc_first = pl.pallas_call(
    _tc_first_body,
    grid=(NBLK,),
    in_specs=[_row_spec, _w_spec, _deg_spec],
    out_specs=_row_spec,
    out_shape=jax.ShapeDtypeStruct((N_NODES, D), jnp.float32),
)

_tc_mid = pl.pallas_call(
    _tc_mid_body,
    grid=(NBLK,),
    in_specs=[_row_spec, _row_spec, _deg_spec, _b_spec, _w_spec],
    out_specs=_row_spec,
    out_shape=jax.ShapeDtypeStruct((N_NODES, D), jnp.float32),
)

_tc_pool = pl.pallas_call(
    _tc_pool_body,
    grid=(NBLK,),
    in_specs=[_row_spec, _row_spec, _deg_spec, _b_spec,
              pl.BlockSpec((D, 1), lambda i: (0, 0)),
              pl.BlockSpec((1, 1), lambda i: (0, 0)),
              pl.BlockSpec((R, 1), lambda i: (i, 0))],
    out_specs=pl.BlockSpec((G, 1), lambda i: (0, 0)),
    out_shape=jax.ShapeDtypeStruct((G, 1), jnp.float32),
    scratch_shapes=[pltpu.VMEM((G, 1), jnp.float32)],
)


def kernel(x, edge_index, batch, W1, b1, W2, b2, W3, b3, Wl, bl):
    row = edge_index[0]
    col = edge_index[1]
    col2 = col.reshape(N_EDGES, 1)
    bases_row, blo, bhi = _tc_counts(col2)
    pos = _tc_pos(col2, bases_row).reshape(N_EDGES)
    row_p = jnp.concatenate([row, jnp.zeros((PADE,), jnp.int32)])
    col_p = jnp.concatenate([col, jnp.full((PADE,), N_PAD, jnp.int32)])
    pos_p = jnp.concatenate([pos, jnp.arange(N_EDGES, EPADDED, dtype=jnp.int32)])
    brow, bcol = _place_sc(row_p.reshape(EPADDED // GC, GC),
                           col_p.reshape(EPADDED // GC, GC),
                           pos_p.reshape(EPADDED // GC, GC))
    blo = blo.reshape(NW, 1, 16)
    bhi = bhi.reshape(NW, 1, 16)
    deg_r = _deg_sc(bcol, blo, bhi)
    deg3 = deg_r.reshape(NW, DROWS * 16)[:, 1:BROWS + 1]
    deg3 = deg3.reshape(N_PAD, 1)[:N_NODES]
    t1 = _tc_first(x, W1, deg3)
    a1 = _spmm_sc(bcol, brow, blo, bhi, t1)
    t2 = _tc_mid(a1, t1, deg3, b1.reshape(1, D), W2)
    a2 = _spmm_sc(bcol, brow, blo, bhi, t2)
    t3 = _tc_mid(a2, t2, deg3, b2.reshape(1, D), W3)
    a3 = _spmm_sc(bcol, brow, blo, bhi, t3)
    out = _tc_pool(a3, t3, deg3, b3.reshape(1, D), Wl,
                   bl.reshape(1, 1), batch.reshape(N_NODES, 1))
    return out


# speedup vs baseline: 1.0684x; 1.0117x over previous
import functools

import jax
import jax.numpy as jnp
from jax import lax
from jax.experimental import pallas as pl
from jax.experimental.pallas import tpu as pltpu
from jax.experimental.pallas import tpu_sc as plsc

N_NODES = 10000
N_EDGES = 320000
D = 128
G = 64

NC = 2
NS = 16
NW = NC * NS

N_PAD = 10240
BROWS = N_PAD // NW
TRASH = BROWS
GC = 128
SG = 256
SG_SHIFT = 8
EPADDED = 327680
PADE = EPADDED - N_EDGES
EPW = EPADDED // NW

_mesh = plsc.VectorSubcoreMesh(core_axis_name="c", subcore_axis_name="s")

R = 1000
NBLK = N_NODES // R
RB = 2000
NBE = N_EDGES // RB


def _bucket_of(col):
    return lax.shift_right_logical(col * 6554, 21)


def _tc_counts_body(col_ref, brow_ref, blo_ref, bhi_ref, cnt_ref):
    i = pl.program_id(0)
    bkt = _bucket_of(col_ref[...])
    gi = lax.broadcasted_iota(jnp.int32, (RB, NW), 1)
    p = (bkt == gi).astype(jnp.float32)
    cs = jnp.sum(p, axis=0, keepdims=True)

    @pl.when(i == 0)
    def _():
        cnt_ref[...] = cs

    @pl.when(i > 0)
    def _():
        cnt_ref[...] += cs

    @pl.when(i == NBE - 1)
    def _():
        cnt = cnt_ref[...]
        ir = lax.broadcasted_iota(jnp.int32, (NW, NW), 0)
        ic = lax.broadcasted_iota(jnp.int32, (NW, NW), 1)
        tri = (ir < ic).astype(jnp.float32)
        eye = (ir == ic).astype(jnp.float32)
        dn = (((1,), (1,)), ((), ()))
        bases = lax.dot_general(cnt, tri, dn,
                                preferred_element_type=jnp.float32, precision=lax.Precision.HIGHEST)
        brow_ref[...] = bases
        bcol = lax.dot_general(eye, bases, dn,
                               preferred_element_type=jnp.float32, precision=lax.Precision.HIGHEST)
        hcol = lax.dot_general(eye, bases + cnt, dn,
                               preferred_element_type=jnp.float32, precision=lax.Precision.HIGHEST)
        blo_ref[...] = jnp.broadcast_to(bcol, (NW, 16)).astype(jnp.int32)
        bhi_ref[...] = jnp.broadcast_to(hcol, (NW, 16)).astype(jnp.int32)


_tc_counts = pl.pallas_call(
    _tc_counts_body,
    grid=(NBE,),
    in_specs=[pl.BlockSpec((RB, 1), lambda i: (i, 0))],
    out_specs=[pl.BlockSpec((1, NW), lambda i: (0, 0)),
               pl.BlockSpec((NW, 16), lambda i: (0, 0)),
               pl.BlockSpec((NW, 16), lambda i: (0, 0))],
    out_shape=[jax.ShapeDtypeStruct((1, NW), jnp.float32),
               jax.ShapeDtypeStruct((NW, 16), jnp.int32),
               jax.ShapeDtypeStruct((NW, 16), jnp.int32)],
    scratch_shapes=[pltpu.VMEM((1, NW), jnp.float32)],
)


def _tc_pos_body(col_ref, brow_ref, pos_ref, carry_ref):
    i = pl.program_id(0)

    @pl.when(i == 0)
    def _():
        carry_ref[...] = jnp.zeros((1, NW), jnp.float32)

    bkt = _bucket_of(col_ref[...])
    gi = lax.broadcasted_iota(jnp.int32, (RB, NW), 1)
    p = (bkt == gi).astype(jnp.float32)
    ri = lax.broadcasted_iota(jnp.int32, (RB, NW), 0)
    pr = p
    sh = 1
    while sh < RB:
        pr = pr + jnp.where(ri >= sh, pltpu.roll(pr, sh, 0), 0.0)
        sh *= 2
    pr = pr - p
    offs = carry_ref[...] + brow_ref[...]
    posf = jnp.sum(p * (pr + offs), axis=1, keepdims=True)
    pos_ref[...] = posf.astype(jnp.int32)
    carry_ref[...] += jnp.sum(p, axis=0, keepdims=True)


_tc_pos = pl.pallas_call(
    _tc_pos_body,
    grid=(NBE,),
    in_specs=[pl.BlockSpec((RB, 1), lambda i: (i, 0)),
              pl.BlockSpec((1, NW), lambda i: (0, 0))],
    out_specs=pl.BlockSpec((RB, 1), lambda i: (i, 0)),
    out_shape=jax.ShapeDtypeStruct((N_EDGES, 1), jnp.int32),
    scratch_shapes=[pltpu.VMEM((1, NW), jnp.float32)],
)


SPR = 8


@functools.partial(
    pl.kernel,
    out_type=(jax.ShapeDtypeStruct((EPADDED,), jnp.int32),
              jax.ShapeDtypeStruct((EPADDED,), jnp.int32)),
    mesh=_mesh,
    scratch_types=[
        pltpu.VMEM((SPR, GC), jnp.int32),
        pltpu.VMEM((SPR, GC), jnp.int32),
        pltpu.VMEM((SPR, GC), jnp.int32),
        pltpu.SemaphoreType.DMA,
    ],
)
def _place_sc(row_hbm, col_hbm, pos_hbm, brow_hbm, bcol_hbm,
              rbuf, cbuf, pbuf, sem):
    c = lax.axis_index("c")
    s = lax.axis_index("s")
    w = s * NC + c
    base = w * (EPW // GC)

    def body(j, _):
        ro = base + j * SPR
        pltpu.sync_copy(row_hbm.at[pl.ds(ro, SPR), :], rbuf)
        pltpu.sync_copy(col_hbm.at[pl.ds(ro, SPR), :], cbuf)
        pltpu.sync_copy(pos_hbm.at[pl.ds(ro, SPR), :], pbuf)
        handles = []
        for k in range(SPR):
            ps = pbuf.at[k]
            handles.append(pltpu.async_copy(rbuf.at[k], brow_hbm.at[ps], sem))
            handles.append(pltpu.async_copy(cbuf.at[k], bcol_hbm.at[ps], sem))
        for h in handles:
            h.wait()
        return 0

    lax.fori_loop(0, EPW // (SPR * GC), body, 0)


def _drain_bounds(blo_hbm, bhi_hbm, w, bbuf, shift=7):
    pltpu.sync_copy(blo_hbm.at[w], bbuf)
    b0 = bbuf[0, pl.ds(0, 16)][0]
    pltpu.sync_copy(bhi_hbm.at[w], bbuf)
    b1 = bbuf[0, pl.ds(0, 16)][0]
    start = lax.shift_left(lax.shift_right_logical(b0, 7), 7)
    nchunk = lax.shift_right_logical(b1 - start + (1 << shift) - 1, shift)
    return start, nchunk


DROWS = BROWS // 16 + 1


@functools.partial(
    pl.kernel,
    out_type=jax.ShapeDtypeStruct((NW, DROWS, 16), jnp.float32),
    mesh=_mesh,
    scratch_types=[
        pltpu.VMEM((GC,), jnp.int32),
        pltpu.VMEM((1, 16), jnp.int32),
        pltpu.VMEM((DROWS + 7, 16), jnp.float32),
    ],
)
def _deg_sc(bcol_hbm, blo_hbm, bhi_hbm, out_hbm, cbuf, bbuf, degl):
    c = lax.axis_index("c")
    s = lax.axis_index("s")
    w = s * NC + c
    lo = w * BROWS
    iota_f = lax.iota(jnp.int32, 16).astype(jnp.float32)
    lo16 = jnp.full((16,), lo, jnp.int32)
    neg16 = jnp.full((16,), -1, jnp.int32)
    brows16 = jnp.full((16,), BROWS, jnp.int32)

    def _zrow(i, _):
        degl[i, pl.ds(0, 16)] = jnp.zeros((16,), jnp.float32)
        return 0

    lax.fori_loop(0, DROWS + 7, _zrow, 0)

    start, nchunk = _drain_bounds(blo_hbm, bhi_hbm, w, bbuf)

    def drain(i, _):
        off = pl.multiple_of(start + i * GC, GC)
        pltpu.sync_copy(bcol_hbm.at[pl.ds(off, GC)], cbuf)

        def vb(v, _):
            cv = cbuf[pl.ds(v * 16, 16)]
            ck = jnp.minimum(jnp.maximum(cv - lo16, neg16), brows16) + 1
            for l in range(16):
                cl = ck[l]
                r = lax.shift_right_logical(cl, 4)
                lane_f = jnp.full((16,), (cl & 15).astype(jnp.float32))
                oh = jnp.maximum(1.0 - jnp.abs(iota_f - lane_f), 0.0)
                degl[r, pl.ds(0, 16)] += oh
            return 0

        lax.fori_loop(0, GC // 16, vb, 0)
        return 0

    lax.fori_loop(0, nchunk, drain, 0)
    pltpu.sync_copy(degl.at[pl.ds(0, DROWS), :], out_hbm.at[w])


@functools.partial(
    pl.kernel,
    out_type=jax.ShapeDtypeStruct((N_PAD, D), jnp.float32),
    mesh=_mesh,
    scratch_types=[
        pltpu.VMEM((SG,), jnp.int32),
        pltpu.VMEM((SG,), jnp.int32),
        pltpu.VMEM((SG,), jnp.int32),
        pltpu.VMEM((SG,), jnp.int32),
        pltpu.VMEM((1, 16), jnp.int32),
        pltpu.VMEM((SG, D), jnp.float32),
        pltpu.VMEM((SG, D), jnp.float32),
        pltpu.VMEM((BROWS + 8, D), jnp.float32),
        pltpu.SemaphoreType.DMA,
        pltpu.SemaphoreType.DMA,
        pltpu.SemaphoreType.DMA,
    ],
)
def _spmm_sc(bcol_hbm, brow_hbm, blo_hbm, bhi_hbm, t_hbm, out_hbm,
             cbuf, rbuf, cbuf2, rbuf2, bbuf, rows_b, rows_b2, acc,
             sem, semb, seml):
    c = lax.axis_index("c")
    s = lax.axis_index("s")
    w = s * NC + c
    lo = w * BROWS
    lo16 = jnp.full((16,), lo, jnp.int32)
    neg16 = jnp.full((16,), -1, jnp.int32)
    brows16 = jnp.full((16,), BROWS, jnp.int32)

    def _zrow(i, _):
        for f in range(D // 16):
            acc[i, pl.ds(f * 16, 16)] = jnp.zeros((16,), jnp.float32)
        return 0

    lax.fori_loop(0, BROWS + 8, _zrow, 0)

    start, nchunk = _drain_bounds(blo_hbm, bhi_hbm, w, bbuf, shift=SG_SHIFT)

    def _issue_gathers(rb, dst, sm):
        return [
            pltpu.async_copy(t_hbm.at[rb.at[pl.ds(k * GC, GC)]],
                             dst.at[pl.ds(k * GC, GC), :], sm)
            for k in range(SG // GC)
        ]

    def _accum(cb, dst):
        def vb(v, _):
            cv = cb[pl.ds(v * 16, 16)]
            ck = jnp.minimum(jnp.maximum(cv - lo16, neg16), brows16) + 1
            for l in range(16):
                cl = ck[l]
                for f in range(D // 16):
                    acc[cl, pl.ds(f * 16, 16)] += dst[v * 16 + l,
                                                      pl.ds(f * 16, 16)]
            return 0

        lax.fori_loop(0, SG // 16, vb, 0)

    def drain(i, _):
        off_a = pl.multiple_of(start + (2 * i) * SG, GC)
        off_b = pl.multiple_of(start + (2 * i + 1) * SG, GC)
        has_b = 2 * i + 1 < nchunk
        pltpu.sync_copy(bcol_hbm.at[pl.ds(off_a, SG)], cbuf)
        pltpu.sync_copy(brow_hbm.at[pl.ds(off_a, SG)], rbuf)
        ha = _issue_gathers(rbuf, rows_b, sem)

        @pl.when(has_b)
        def _():
            hlc = pltpu.async_copy(bcol_hbm.at[pl.ds(off_b, SG)], cbuf2, seml)
            hlr = pltpu.async_copy(brow_hbm.at[pl.ds(off_b, SG)], rbuf2, seml)
            hlc.wait()
            hlr.wait()
            _issue_gathers(rbuf2, rows_b2, semb)

        for h in ha:
            h.wait()
        _accum(cbuf, rows_b)

        @pl.when(has_b)
        def _():
            for k in range(SG // GC):
                pltpu.make_async_copy(
                    t_hbm.at[rbuf2.at[pl.ds(k * GC, GC)]],
                    rows_b2.at[pl.ds(k * GC, GC), :], semb).wait()
            _accum(cbuf2, rows_b2)

        return 0

    lax.fori_loop(0, lax.shift_right_logical(nchunk + 1, 1), drain, 0)
    pltpu.sync_copy(acc.at[pl.ds(1, BROWS), :],
                    out_hbm.at[pl.ds(lo, BROWS), :])


def _dinv_of(deg_ref):
    d = deg_ref[...] + 1.0
    y = lax.rsqrt(d)
    y = y * (1.5 - 0.5 * d * y * y)
    y = y * (1.5 - 0.5 * d * y * y)
    return y


def _mm(a, w):
    return jnp.dot(a, w, preferred_element_type=jnp.float32)


def _tc_first_body(x_ref, w_ref, deg_ref, out_ref):
    dinv = _dinv_of(deg_ref)
    t = _mm(x_ref[...], w_ref[...])
    out_ref[...] = t * dinv


def _tc_mid_body(a_ref, tp_ref, deg_ref, b_ref, w_ref, out_ref):
    dinv = _dinv_of(deg_ref)
    a = a_ref[...] + tp_ref[...]
    h = jnp.maximum(a * dinv + b_ref[...], 0.0)
    out_ref[...] = _mm(h, w_ref[...]) * dinv


def _tc_pool_body(a_ref, tp_ref, deg_ref, b_ref, wl_ref, bl_ref, batch_ref,
                  out_ref, cnt_ref):
    i = pl.program_id(0)
    dinv = _dinv_of(deg_ref)
    h = (a_ref[...] + tp_ref[...]) * dinv + b_ref[...]
    y = _mm(h, wl_ref[...])
    gids = lax.broadcasted_iota(jnp.int32, (R, G), 1)
    p = (batch_ref[...] == gids).astype(jnp.float32)
    dn = (((0,), (0,)), ((), ()))
    ysum = lax.dot_general(p, y, dn, preferred_element_type=jnp.float32, precision=lax.Precision.HIGHEST)
    csum = lax.dot_general(p, jnp.ones((R, 1), jnp.float32), dn,
                           preferred_element_type=jnp.float32, precision=lax.Precision.HIGHEST)

    @pl.when(i == 0)
    def _():
        out_ref[...] = ysum
        cnt_ref[...] = csum

    @pl.when(i > 0)
    def _():
        out_ref[...] += ysum
        cnt_ref[...] += csum

    @pl.when(i == NBLK - 1)
    def _():
        out_ref[...] = out_ref[...] / jnp.maximum(cnt_ref[...], 1.0) + bl_ref[...]


_deg_spec = pl.BlockSpec((R, 1), lambda i: (i, 0))
_w_spec = pl.BlockSpec((D, D), lambda i: (0, 0))
_b_spec = pl.BlockSpec((1, D), lambda i: (0, 0))
_row_spec = pl.BlockSpec((R, D), lambda i: (i, 0))

_tc_first = pl.pallas_call(
    _tc_first_body,
    grid=(NBLK,),
    in_specs=[_row_spec, _w_spec, _deg_spec],
    out_specs=_row_spec,
    out_shape=jax.ShapeDtypeStruct((N_NODES, D), jnp.float32),
)

_tc_mid = pl.pallas_call(
    _tc_mid_body,
    grid=(NBLK,),
    in_specs=[_row_spec, _row_spec, _deg_spec, _b_spec, _w_spec],
    out_specs=_row_spec,
    out_shape=jax.ShapeDtypeStruct((N_NODES, D), jnp.float32),
)

_tc_pool = pl.pallas_call(
    _tc_pool_body,
    grid=(NBLK,),
    in_specs=[_row_spec, _row_spec, _deg_spec, _b_spec,
              pl.BlockSpec((D, 1), lambda i: (0, 0)),
              pl.BlockSpec((1, 1), lambda i: (0, 0)),
              pl.BlockSpec((R, 1), lambda i: (i, 0))],
    out_specs=pl.BlockSpec((G, 1), lambda i: (0, 0)),
    out_shape=jax.ShapeDtypeStruct((G, 1), jnp.float32),
    scratch_shapes=[pltpu.VMEM((G, 1), jnp.float32)],
)


def kernel(x, edge_index, batch, W1, b1, W2, b2, W3, b3, Wl, bl):
    row = edge_index[0]
    col = edge_index[1]
    col2 = col.reshape(N_EDGES, 1)
    bases_row, blo, bhi = _tc_counts(col2)
    pos = _tc_pos(col2, bases_row).reshape(N_EDGES)
    row_p = jnp.concatenate([row, jnp.zeros((PADE,), jnp.int32)])
    col_p = jnp.concatenate([col, jnp.full((PADE,), N_PAD, jnp.int32)])
    pos_p = jnp.concatenate([pos, jnp.arange(N_EDGES, EPADDED, dtype=jnp.int32)])
    brow, bcol = _place_sc(row_p.reshape(EPADDED // GC, GC),
                           col_p.reshape(EPADDED // GC, GC),
                           pos_p.reshape(EPADDED // GC, GC))
    blo = blo.reshape(NW, 1, 16)
    bhi = bhi.reshape(NW, 1, 16)
    deg_r = _deg_sc(bcol, blo, bhi)
    deg3 = deg_r.reshape(NW, DROWS * 16)[:, 1:BROWS + 1]
    deg3 = deg3.reshape(N_PAD, 1)[:N_NODES]
    t1 = _tc_first(x, W1, deg3)
    a1 = _spmm_sc(bcol, brow, blo, bhi, t1)
    t2 = _tc_mid(a1, t1, deg3, b1.reshape(1, D), W2)
    a2 = _spmm_sc(bcol, brow, blo, bhi, t2)
    t3 = _tc_mid(a2, t2, deg3, b2.reshape(1, D), W3)
    a3 = _spmm_sc(bcol, brow, blo, bhi, t3)
    out = _tc_pool(a3, t3, deg3, b3.reshape(1, D), Wl,
                   bl.reshape(1, 1), batch.reshape(N_NODES, 1))
    return out
